# Initial kernel scaffold; baseline (speedup 1.0000x reference)
import jax
import jax.numpy as jnp
from jax.experimental import pallas as pl


def kernel(x, edge_index, W, b):
    raise NotImplementedError("write your pallas kernel here")


# speedup vs baseline: 2336.3654x; 2336.3654x over previous
import functools

import jax
import jax.numpy as jnp
from jax import lax
from jax.experimental import pallas as pl
from jax.experimental.pallas import tpu as pltpu
from jax.experimental.pallas import tpu_sc as plsc

NTILES = 32
CH = 256
LANES = 16


def _sc_maxdiff(x3, idxb, cpt, n_pad, k_deg, nblk):
    mesh = plsc.VectorSubcoreMesh(core_axis_name="c", subcore_axis_name="s")
    groups = CH // LANES

    @functools.partial(
        pl.kernel,
        out_type=jax.ShapeDtypeStruct((NTILES, cpt, n_pad), jnp.float32),
        mesh=mesh,
        compiler_params=pltpu.CompilerParams(
            needs_layout_passes=False,
            use_tc_tiling_on_sc=False,
        ),
        scratch_types=[
            pltpu.VMEM((cpt * n_pad,), jnp.float32),
            pltpu.VMEM((cpt, n_pad), jnp.float32),
            pltpu.VMEM((k_deg, CH), jnp.int32),
            pltpu.VMEM((k_deg, CH), jnp.int32),
        ],
    )
    def sc_kernel(x_hbm, idx_hbm, m_hbm, table_v, out_v, i0_v, i1_v):
        wid = lax.axis_index("s") * 2 + lax.axis_index("c")
        pltpu.sync_copy(x_hbm.at[wid], table_v)
        coffs = [jnp.full((LANES,), c * n_pad, jnp.int32) for c in range(cpt)]

        def blk_body(blk, carry):
            pltpu.sync_copy(idx_hbm.at[0, blk], i0_v)
            pltpu.sync_copy(idx_hbm.at[1, blk], i1_v)

            def g_body(g, carry2):
                base = g * LANES
                i0 = i0_v[0, pl.ds(base, LANES)]
                i1 = i1_v[0, pl.ds(base, LANES)]
                accs = []
                for c in range(cpt):
                    a = plsc.load_gather(table_v, [i0 + coffs[c]])
                    b2 = plsc.load_gather(table_v, [i1 + coffs[c]])
                    accs.append(a - b2)
                for kk in range(1, k_deg):
                    i0 = i0_v[kk, pl.ds(base, LANES)]
                    i1 = i1_v[kk, pl.ds(base, LANES)]
                    for c in range(cpt):
                        a = plsc.load_gather(table_v, [i0 + coffs[c]])
                        b2 = plsc.load_gather(table_v, [i1 + coffs[c]])
                        accs[c] = jnp.maximum(accs[c], a - b2)
                nb = blk * CH + base
                for c in range(cpt):
                    out_v[c, pl.ds(nb, LANES)] = accs[c]
                return carry2

            lax.fori_loop(0, groups, g_body, 0)
            return carry

        lax.fori_loop(0, nblk, blk_body, 0)
        pltpu.sync_copy(out_v, m_hbm.at[wid])

    return sc_kernel(x3, idxb)


def _tc_conv(xf, m, We, Wo, b2, c, n_pad):
    bn = 1024
    grid = (n_pad // bn,)

    def body(x_ref, m_ref, we_ref, wo_ref, b_ref, y_ref):
        acc = jnp.dot(we_ref[...], x_ref[...], preferred_element_type=jnp.float32)
        acc += jnp.dot(wo_ref[...], m_ref[...], preferred_element_type=jnp.float32)
        y_ref[...] = jnp.maximum(acc + b_ref[...], 0.0)

    return pl.pallas_call(
        body,
        grid=grid,
        in_specs=[
            pl.BlockSpec((c, bn), lambda i: (0, i)),
            pl.BlockSpec((c, bn), lambda i: (0, i)),
            pl.BlockSpec((c, c), lambda i: (0, 0)),
            pl.BlockSpec((c, c), lambda i: (0, 0)),
            pl.BlockSpec((c, 1), lambda i: (0, 0)),
        ],
        out_specs=pl.BlockSpec((c, bn), lambda i: (0, i)),
        out_shape=jax.ShapeDtypeStruct((c, n_pad), jnp.float32),
    )(xf, m, We, Wo, b2)


def kernel(x, edge_index, W, b):
    B, C, N, _ = x.shape
    K = edge_index.shape[-1]
    cpt = C // NTILES
    n_pad = ((N + CH - 1) // CH) * CH
    nblk = n_pad // CH

    xf = x.reshape(C, N)
    xp = jnp.pad(xf, ((0, 0), (0, n_pad - N)))
    x3 = xp.reshape(NTILES, cpt * n_pad)

    ei = edge_index.reshape(2, N, K)
    eip = jnp.pad(ei, ((0, 0), (0, n_pad - N), (0, 0)))
    idxb = eip.reshape(2, nblk, CH, K).transpose(0, 1, 3, 2)

    m3 = _sc_maxdiff(x3, idxb, cpt, n_pad, K, nblk)
    m = m3.reshape(C, n_pad)

    We = W[:, 0::2]
    Wo = W[:, 1::2]
    y = _tc_conv(xp, m, We, Wo, b.reshape(C, 1), C, n_pad)
    return y[:, :N].reshape(x.shape)


# speedup vs baseline: 3405.3626x; 1.4575x over previous
import functools

import jax
import jax.numpy as jnp
from jax import lax
from jax.experimental import pallas as pl
from jax.experimental.pallas import tpu as pltpu
from jax.experimental.pallas import tpu_sc as plsc

NTILES = 32
NGRP = 4
CGRP = 8
CH = 256
LANES = 16


def _sc_maxdiff(xw, idxb, n_pad, k_deg):
    pairs = xw.shape[1] // n_pad
    n_sub = n_pad // NGRP
    blocks = n_sub // CH
    groups = CH // LANES
    mesh = plsc.VectorSubcoreMesh(core_axis_name="c", subcore_axis_name="s")

    @functools.partial(
        pl.kernel,
        out_type=jax.ShapeDtypeStruct((NTILES, pairs, n_sub), jnp.int32),
        mesh=mesh,
        compiler_params=pltpu.CompilerParams(
            needs_layout_passes=False,
            use_tc_tiling_on_sc=False,
        ),
        scratch_types=[
            pltpu.VMEM((pairs * n_pad,), jnp.int32),
            pltpu.VMEM((pairs, n_sub), jnp.int32),
            pltpu.VMEM((k_deg, CH), jnp.int32),
            pltpu.VMEM((k_deg, CH), jnp.int32),
        ],
    )
    def sc_kernel(x_hbm, idx_hbm, m_hbm, table_v, out_v, i0_v, i1_v):
        wid = lax.axis_index("s") * 2 + lax.axis_index("c")
        cg = wid // NGRP
        ng = wid % NGRP
        pltpu.sync_copy(x_hbm.at[cg], table_v)
        poffs = [jnp.full((LANES,), p * n_pad, jnp.int32) for p in range(pairs)]

        def blk_body(j, carry):
            blk = ng * blocks + j
            pltpu.sync_copy(idx_hbm.at[0, blk], i0_v)
            pltpu.sync_copy(idx_hbm.at[1, blk], i1_v)

            def g_body(g, carry2):
                base = g * LANES
                accs = []
                for kk in range(k_deg):
                    i0 = i0_v[kk, pl.ds(base, LANES)]
                    i1 = i1_v[kk, pl.ds(base, LANES)]
                    for p in range(pairs):
                        a = plsc.load_gather(table_v, [i0 + poffs[p]])
                        b2 = plsc.load_gather(table_v, [i1 + poffs[p]])
                        d = plsc.bitcast(a, jnp.bfloat16) - plsc.bitcast(b2, jnp.bfloat16)
                        if kk == 0:
                            accs.append(d)
                        else:
                            accs[p] = jnp.maximum(accs[p], d)
                loc = j * CH + base
                for p in range(pairs):
                    out_v[p, pl.ds(loc, LANES)] = plsc.bitcast(accs[p], jnp.int32)
                return carry2

            lax.fori_loop(0, groups, g_body, 0)
            return carry

        lax.fori_loop(0, blocks, blk_body, 0)
        pltpu.sync_copy(out_v, m_hbm.at[wid])

    return sc_kernel(xw, idxb)


def _tc_conv(xf, m, We, Wo, b2, c, n_pad):
    bn = 1024
    grid = (n_pad // bn,)

    def body(x_ref, m_ref, we_ref, wo_ref, b_ref, y_ref):
        acc = jnp.dot(we_ref[...], x_ref[...], preferred_element_type=jnp.float32)
        acc += jnp.dot(
            wo_ref[...],
            m_ref[...].astype(jnp.float32),
            preferred_element_type=jnp.float32,
        )
        y_ref[...] = jnp.maximum(acc + b_ref[...], 0.0)

    return pl.pallas_call(
        body,
        grid=grid,
        in_specs=[
            pl.BlockSpec((c, bn), lambda i: (0, i)),
            pl.BlockSpec((c, bn), lambda i: (0, i)),
            pl.BlockSpec((c, c), lambda i: (0, 0)),
            pl.BlockSpec((c, c), lambda i: (0, 0)),
            pl.BlockSpec((c, 1), lambda i: (0, 0)),
        ],
        out_specs=pl.BlockSpec((c, bn), lambda i: (0, i)),
        out_shape=jax.ShapeDtypeStruct((c, n_pad), jnp.float32),
    )(xf, m, We, Wo, b2)


def kernel(x, edge_index, W, b):
    B, C, N, _ = x.shape
    K = edge_index.shape[-1]
    n_pad = ((N + CH - 1) // CH) * CH
    nblk = n_pad // CH
    n_sub = n_pad // NGRP
    pairs = C // (2 * CGRP)

    xf = x.reshape(C, N)
    xp = jnp.pad(xf, ((0, 0), (0, n_pad - N)))

    xb = xp.astype(jnp.bfloat16)
    xwords = lax.bitcast_convert_type(
        xb.reshape(C // 2, 2, n_pad).transpose(0, 2, 1), jnp.int32
    )
    xw = xwords.reshape(CGRP, pairs * n_pad)

    ei = edge_index.reshape(2, N, K)
    eip = jnp.pad(ei, ((0, 0), (0, n_pad - N), (0, 0)))
    idxb = eip.reshape(2, nblk, CH, K).transpose(0, 1, 3, 2)

    mw = _sc_maxdiff(xw, idxb, n_pad, K)
    mb = lax.bitcast_convert_type(mw, jnp.bfloat16)
    m = (
        mb.reshape(CGRP, NGRP, pairs, n_sub, 2)
        .transpose(0, 2, 4, 1, 3)
        .reshape(C, n_pad)
    )

    We = W[:, 0::2]
    Wo = W[:, 1::2]
    y = _tc_conv(xp, m, We, Wo, b.reshape(C, 1), C, n_pad)
    return y[:, :N].reshape(x.shape)


# speedup vs baseline: 4297.6596x; 1.2620x over previous
import functools

import jax
import jax.numpy as jnp
from jax import lax
from jax.experimental import pallas as pl
from jax.experimental.pallas import tpu as pltpu
from jax.experimental.pallas import tpu_sc as plsc

NTILES = 32
NGRP = 2
CGRP = 16
CH = 256
LANES = 16


def _sc_maxdiff(xw, idxb, n_tab, n_pad, k_deg):
    pairs = xw.shape[1] // n_tab
    n_sub = n_pad // NGRP
    blocks = n_sub // CH
    groups = CH // LANES
    mesh = plsc.VectorSubcoreMesh(core_axis_name="c", subcore_axis_name="s")

    @functools.partial(
        pl.kernel,
        out_type=jax.ShapeDtypeStruct((2 * CGRP * pairs, n_pad), jnp.float32),
        mesh=mesh,
        compiler_params=pltpu.CompilerParams(
            needs_layout_passes=False,
            use_tc_tiling_on_sc=False,
        ),
        scratch_types=[
            pltpu.VMEM((pairs * n_tab,), jnp.int32),
            pltpu.VMEM((2 * pairs, n_sub), jnp.float32),
            pltpu.VMEM((k_deg, CH), jnp.int32),
            pltpu.VMEM((k_deg, CH), jnp.int32),
            pltpu.VMEM((k_deg, CH), jnp.int32),
            pltpu.VMEM((k_deg, CH), jnp.int32),
            pltpu.SemaphoreType.DMA,
            pltpu.SemaphoreType.DMA,
        ],
    )
    def sc_kernel(x_hbm, idx_hbm, m_hbm, table_v, out_v,
                  i0a, i1a, i0b, i1b, sem_a, sem_b):
        wid = lax.axis_index("s") * 2 + lax.axis_index("c")
        cg = wid // NGRP
        ng = wid % NGRP
        pltpu.sync_copy(x_hbm.at[cg], table_v)
        poffs = [jnp.full((LANES,), p * n_tab, jnp.int32) for p in range(pairs)]
        blk0 = ng * blocks

        def issue(blk, d0, d1, sem):
            pltpu.async_copy(idx_hbm.at[0, blk], d0, sem)
            pltpu.async_copy(idx_hbm.at[1, blk], d1, sem)

        def drain(blk, d0, d1, sem):
            pltpu.make_async_copy(idx_hbm.at[0, blk], d0, sem).wait()
            pltpu.make_async_copy(idx_hbm.at[1, blk], d1, sem).wait()

        def compute(j, b0, b1):
            def g_body(g, carry):
                base = g * LANES
                accs = []
                for kk in range(k_deg):
                    i0 = b0[kk, pl.ds(base, LANES)]
                    i1 = b1[kk, pl.ds(base, LANES)]
                    for p in range(pairs):
                        a = plsc.load_gather(table_v, [i0 + poffs[p]])
                        b2 = plsc.load_gather(table_v, [i1 + poffs[p]])
                        d = plsc.bitcast(a, jnp.bfloat16) - plsc.bitcast(b2, jnp.bfloat16)
                        if kk == 0:
                            accs.append(d)
                        else:
                            accs[p] = jnp.maximum(accs[p], d)
                loc = j * CH + base
                for p in range(pairs):
                    lo, hi = plsc.unpack(accs[p], format=plsc.PackFormat.INTERLEAVED)
                    out_v[2 * p, pl.ds(loc, LANES)] = lo
                    out_v[2 * p + 1, pl.ds(loc, LANES)] = hi
                return carry

            lax.fori_loop(0, groups, g_body, 0)

        issue(blk0, i0a, i1a, sem_a)

        def super_body(it, carry):
            ja = 2 * it
            jb = 2 * it + 1
            issue(blk0 + jb, i0b, i1b, sem_b)
            drain(blk0 + ja, i0a, i1a, sem_a)
            compute(ja, i0a, i1a)

            @pl.when(it + 1 < blocks // 2)
            def _():
                issue(blk0 + ja + 2, i0a, i1a, sem_a)

            drain(blk0 + jb, i0b, i1b, sem_b)
            compute(jb, i0b, i1b)
            return carry

        lax.fori_loop(0, blocks // 2, super_body, 0)
        pltpu.sync_copy(
            out_v,
            m_hbm.at[pl.ds(cg * 2 * pairs, 2 * pairs), pl.ds(ng * n_sub, n_sub)],
        )

    return sc_kernel(xw, idxb)


def _tc_conv(xf, m, We, Wo, b2, c, n):
    def body(x_ref, m_ref, we_ref, wo_ref, b_ref, y_ref):
        acc = jnp.dot(we_ref[...], x_ref[...], preferred_element_type=jnp.float32)
        acc += jnp.dot(wo_ref[...], m_ref[:, :n], preferred_element_type=jnp.float32)
        y_ref[...] = jnp.maximum(acc + b_ref[...], 0.0)

    return pl.pallas_call(
        body,
        out_shape=jax.ShapeDtypeStruct((c, n), jnp.float32),
    )(xf, m, We, Wo, b2)


def kernel(x, edge_index, W, b):
    B, C, N, _ = x.shape
    K = edge_index.shape[-1]
    n_pad = ((N + (NGRP * CH) - 1) // (NGRP * CH)) * (NGRP * CH)
    nblk = n_pad // CH
    pairs = C // (2 * CGRP)

    xf = x.reshape(C, N)
    xb = xf.astype(jnp.bfloat16)
    xwords = lax.bitcast_convert_type(
        xb.reshape(C // 2, 2, N).transpose(0, 2, 1), jnp.int32
    )
    xw = xwords.reshape(CGRP, pairs * N)

    ei = edge_index.reshape(2, N, K)
    eip = jnp.pad(ei, ((0, 0), (0, n_pad - N), (0, 0)))
    idxb = eip.reshape(2, nblk, CH, K).transpose(0, 1, 3, 2)

    m = _sc_maxdiff(xw, idxb, N, n_pad, K)

    We = W[:, 0::2]
    Wo = W[:, 1::2]
    y = _tc_conv(xf, m, We, Wo, b.reshape(C, 1), C, N)
    return y.reshape(x.shape)
